# Initial kernel scaffold; baseline (speedup 1.0000x reference)
import functools

import jax
import jax.numpy as jnp
from jax import lax
from jax.experimental import pallas as pl
from jax.experimental.pallas import tpu as pltpu
from jax.experimental.pallas import tpu_sc as plsc

N = 10000
E = 320000
D_IN = 128
D_HID = 64
N_GRAPHS = 64
N_CLASSES = 3

NC = 2
NS = 16
NW = NC * NS
LANES = 16

N_ACC = 10016
ROWS_PER_TILE = N_ACC // NS
CHUNK = 128
E_RAW = E + N
CHUNKS_PER_TILE = -(-E_RAW // (NW * CHUNK))
EDGES_PER_TILE = CHUNKS_PER_TILE * CHUNK
E_PAD = NW * EDGES_PER_TILE


def _pre_body(x_ref, w_ref, asrc_ref, adst_ref, h_ref, av_ref, bv_ref):
    h = jnp.dot(x_ref[...], w_ref[...], preferred_element_type=jnp.float32)
    h_ref[...] = h
    av_ref[...] = jnp.dot(h, asrc_ref[...], preferred_element_type=jnp.float32)
    bv_ref[...] = jnp.dot(h, adst_ref[...], preferred_element_type=jnp.float32)


def _pre(x, w, att_src, att_dst):
    return pl.pallas_call(
        _pre_body,
        out_shape=(
            jax.ShapeDtypeStruct((N, D_HID), jnp.float32),
            jax.ShapeDtypeStruct((N, 1), jnp.float32),
            jax.ShapeDtypeStruct((N, 1), jnp.float32),
        ),
    )(x, w, att_src.reshape(D_HID, 1), att_dst.reshape(D_HID, 1))


def _edge_body(src_hbm, dst_hbm, as_hbm, ad_hbm, h_hbm, zacc_hbm,
               acc_out, den_out,
               asv, adv, srcv, dstv, eav, rows, denv, acc_sh, sem):
    c = lax.axis_index("c")
    s = lax.axis_index("s")
    wid = s * NC + c

    rsl = pl.ds(s * ROWS_PER_TILE, ROWS_PER_TILE)
    pltpu.sync_copy(zacc_hbm.at[rsl], acc_sh.at[rsl])

    pltpu.sync_copy(as_hbm, asv)
    pltpu.sync_copy(ad_hbm, adv)

    def _zden(i, carry):
        denv[pl.ds(i * LANES, LANES)] = jnp.zeros((LANES,), jnp.float32)
        return carry
    lax.fori_loop(0, N_ACC // LANES, _zden, 0)

    plsc.subcore_barrier()

    base0 = wid * EDGES_PER_TILE

    def _chunk(ci, carry):
        base = base0 + ci * CHUNK
        pltpu.sync_copy(src_hbm.at[pl.ds(base, CHUNK)], srcv)
        pltpu.sync_copy(dst_hbm.at[pl.ds(base, CHUNK)], dstv)
        cp = pltpu.async_copy(h_hbm.at[srcv], rows, sem)

        def _ea(j, cc):
            sl = pl.ds(j * LANES, LANES)
            sv = srcv[sl]
            dv = dstv[sl]
            a = plsc.load_gather(asv, [sv]) + plsc.load_gather(adv, [dv])
            a = jnp.where(a >= 0.0, a, a * jnp.float32(0.2))
            e = jnp.exp(a)
            eav[sl] = e
            plsc.addupdate_scatter(denv, [dv], e)
            return cc
        lax.fori_loop(0, CHUNK // LANES, _ea, 0)

        cp.wait()

        def _scale(r, cc):
            w = plsc.load_gather(eav, [jnp.full((LANES,), r, jnp.int32)])
            for k in range(D_HID // LANES):
                sl = pl.ds(k * LANES, LANES)
                rows[r, sl] = rows[r, sl] * w
            return cc
        lax.fori_loop(0, CHUNK, _scale, 0)

        pltpu.sync_copy(rows, acc_sh.at[dstv], add=True)
        return carry

    lax.fori_loop(0, CHUNKS_PER_TILE, _chunk, 0)

    pltpu.sync_copy(denv, den_out.at[wid])
    plsc.subcore_barrier()
    pltpu.sync_copy(acc_sh.at[rsl], acc_out.at[c, rsl])


def _edge_phase(src_pad, dst_pad, as_pad, ad_pad, h_pad, zacc):
    k = pl.kernel(
        _edge_body,
        out_type=(
            jax.ShapeDtypeStruct((NC, N_ACC, D_HID), jnp.float32),
            jax.ShapeDtypeStruct((NW, N_ACC), jnp.float32),
        ),
        mesh=plsc.VectorSubcoreMesh(core_axis_name="c", subcore_axis_name="s"),
        scratch_types=[
            pltpu.VMEM((N_ACC,), jnp.float32),
            pltpu.VMEM((N_ACC,), jnp.float32),
            pltpu.VMEM((CHUNK,), jnp.int32),
            pltpu.VMEM((CHUNK,), jnp.int32),
            pltpu.VMEM((CHUNK,), jnp.float32),
            pltpu.VMEM((CHUNK, D_HID), jnp.float32),
            pltpu.VMEM((N_ACC,), jnp.float32),
            pltpu.VMEM_SHARED((N_ACC, D_HID), jnp.float32),
            pltpu.SemaphoreType.DMA,
        ],
    )
    return k(src_pad, dst_pad, as_pad, ad_pad, h_pad, zacc)


def _post_body(a0_ref, a1_ref, den_ref, bias_ref, batch_ref, fcw_ref, fcb_ref,
               out_ref):
    acc = a0_ref[...] + a1_ref[...]
    den = jnp.sum(den_ref[...], axis=0)
    node = acc / (den + 1e-16)[:, None] + bias_ref[...]
    gids = lax.broadcasted_iota(jnp.int32, (1, N_GRAPHS), 1)
    p = (batch_ref[...] == gids).astype(jnp.float32)
    sums = lax.dot_general(p, node, (((0,), (0,)), ((), ())),
                           preferred_element_type=jnp.float32)
    counts = jnp.sum(p, axis=0)
    feats = sums / jnp.maximum(counts, 1.0)[:, None]
    logits = jnp.dot(feats, fcw_ref[...],
                     preferred_element_type=jnp.float32) + fcb_ref[...]
    m = jnp.max(logits, axis=1, keepdims=True)
    lse = jnp.log(jnp.sum(jnp.exp(logits - m), axis=1, keepdims=True)) + m
    out_ref[...] = logits - lse


def _post(a0, a1, den, bias, batch2d, fc_w, fc_b):
    return pl.pallas_call(
        _post_body,
        out_shape=jax.ShapeDtypeStruct((N_GRAPHS, N_CLASSES), jnp.float32),
    )(a0, a1, den, bias.reshape(1, D_HID), batch2d, fc_w,
      fc_b.reshape(1, N_CLASSES))


def kernel(x, edge_index, batch, W, att_src, att_dst, bias, fc_W, fc_b):
    h, av, bv = _pre(x, W, att_src, att_dst)

    loop = jnp.arange(N, dtype=jnp.int32)
    padi = jnp.full((E_PAD - E_RAW,), N, dtype=jnp.int32)
    src_pad = jnp.concatenate([edge_index[0], loop, padi])
    dst_pad = jnp.concatenate([edge_index[1], loop, padi])

    zrow = jnp.zeros((N_ACC - N,), jnp.float32)
    as_pad = jnp.concatenate([av.reshape(-1), zrow])
    ad_pad = jnp.concatenate([bv.reshape(-1), zrow])
    h_pad = jnp.concatenate([h, jnp.zeros((N_ACC - N, D_HID), jnp.float32)])
    zacc = jnp.zeros((N_ACC, D_HID), jnp.float32)

    acc_parts, den_parts = _edge_phase(src_pad, dst_pad, as_pad, ad_pad,
                                       h_pad, zacc)

    return _post(acc_parts[0, :N], acc_parts[1, :N], den_parts[:, :N],
                 bias, batch.reshape(N, 1), fc_W, fc_b)


# speedup vs baseline: 29.4846x; 29.4846x over previous
import functools

import jax
import jax.numpy as jnp
from jax import lax
from jax.experimental import pallas as pl
from jax.experimental.pallas import tpu as pltpu
from jax.experimental.pallas import tpu_sc as plsc

N = 10000
E = 320000
D_IN = 128
D_HID = 64
N_GRAPHS = 64
N_CLASSES = 3

NC = 2
NS = 16
NW = NC * NS
LANES = 16

N_ACC = 10112
ROWS_PER_TILE = N_ACC // NS
CHUNK = 128
E_RAW = E + N
CHUNKS_PER_TILE = -(-E_RAW // (NW * CHUNK))
EDGES_PER_TILE = CHUNKS_PER_TILE * CHUNK
E_PAD = NW * EDGES_PER_TILE


def _pre_body(x_ref, w_ref, asrc_ref, adst_ref, h_ref, av_ref, bv_ref):
    h = jnp.dot(x_ref[...], w_ref[...], preferred_element_type=jnp.float32)
    h_ref[...] = h
    av_ref[...] = jnp.dot(h, asrc_ref[...], preferred_element_type=jnp.float32)
    bv_ref[...] = jnp.dot(h, adst_ref[...], preferred_element_type=jnp.float32)


def _pre(x, w, att_src, att_dst):
    return pl.pallas_call(
        _pre_body,
        out_shape=(
            jax.ShapeDtypeStruct((N, D_HID), jnp.float32),
            jax.ShapeDtypeStruct((N, 1), jnp.float32),
            jax.ShapeDtypeStruct((N, 1), jnp.float32),
        ),
    )(x, w, att_src.reshape(D_HID, 1), att_dst.reshape(D_HID, 1))


def _edge_body(src_hbm, dst_hbm, as_hbm, ad_hbm, h_hbm, zacc_hbm,
               acc_out, den_out,
               asv, adv, srcv, dstv, eav, rows, denv, acc_sh, sem):
    c = lax.axis_index("c")
    s = lax.axis_index("s")
    wid = s * NC + c

    rsl = pl.ds(s * ROWS_PER_TILE, ROWS_PER_TILE)
    pltpu.sync_copy(zacc_hbm.at[rsl], acc_sh.at[rsl])

    pltpu.sync_copy(as_hbm, asv)
    pltpu.sync_copy(ad_hbm, adv)

    def _zden(i, carry):
        denv[pl.ds(i * LANES, LANES)] = jnp.zeros((LANES,), jnp.float32)
        return carry
    lax.fori_loop(0, N_ACC // LANES, _zden, 0)

    plsc.subcore_barrier()

    base0 = wid * EDGES_PER_TILE

    def _chunk(ci, carry):
        base = base0 + ci * CHUNK
        pltpu.sync_copy(src_hbm.at[pl.ds(base, CHUNK)], srcv)
        pltpu.sync_copy(dst_hbm.at[pl.ds(base, CHUNK)], dstv)
        cp = pltpu.async_copy(h_hbm.at[srcv], rows, sem)

        def _ea(j, cc):
            sl = pl.ds(j * LANES, LANES)
            sv = srcv[sl]
            dv = dstv[sl]
            a = plsc.load_gather(asv, [sv]) + plsc.load_gather(adv, [dv])
            a = jnp.where(a >= 0.0, a, a * jnp.float32(0.2))
            e = jnp.exp(a)
            eav[sl] = e
            plsc.addupdate_scatter(denv, [dv], e)
            return cc
        lax.fori_loop(0, CHUNK // LANES, _ea, 0)

        cp.wait()

        def _scale(r, cc):
            w = plsc.load_gather(eav, [jnp.full((LANES,), r, jnp.int32)])
            for k in range(D_HID // LANES):
                sl = pl.ds(k * LANES, LANES)
                rows[r, sl] = rows[r, sl] * w
            return cc
        lax.fori_loop(0, CHUNK, _scale, 0)

        pltpu.sync_copy(rows, acc_sh.at[dstv], add=True)
        return carry

    lax.fori_loop(0, CHUNKS_PER_TILE, _chunk, 0)

    pltpu.sync_copy(denv, den_out.at[wid])
    plsc.subcore_barrier()
    pltpu.sync_copy(acc_sh.at[rsl], acc_out.at[c, rsl])


def _edge_phase(src_pad, dst_pad, as_pad, ad_pad, h_pad, zacc):
    k = pl.kernel(
        _edge_body,
        out_type=(
            jax.ShapeDtypeStruct((NC, N_ACC, D_HID), jnp.float32),
            jax.ShapeDtypeStruct((NW, N_ACC), jnp.float32),
        ),
        mesh=plsc.VectorSubcoreMesh(core_axis_name="c", subcore_axis_name="s"),
        compiler_params=pltpu.CompilerParams(needs_layout_passes=False,
                                             use_tc_tiling_on_sc=False),
        scratch_types=[
            pltpu.VMEM((N_ACC,), jnp.float32),
            pltpu.VMEM((N_ACC,), jnp.float32),
            pltpu.VMEM((CHUNK,), jnp.int32),
            pltpu.VMEM((CHUNK,), jnp.int32),
            pltpu.VMEM((CHUNK,), jnp.float32),
            pltpu.VMEM((CHUNK, D_HID), jnp.float32),
            pltpu.VMEM((N_ACC,), jnp.float32),
            pltpu.VMEM_SHARED((N_ACC, D_HID), jnp.float32),
            pltpu.SemaphoreType.DMA,
        ],
    )
    return k(src_pad, dst_pad, as_pad, ad_pad, h_pad, zacc)


def _post_body(a0_ref, a1_ref, den_ref, bias_ref, batch_ref, fcw_ref, fcb_ref,
               out_ref):
    acc = a0_ref[...] + a1_ref[...]
    den = jnp.sum(den_ref[...], axis=0)
    node = acc / (den + 1e-16)[:, None] + bias_ref[...]
    gids = lax.broadcasted_iota(jnp.int32, (1, N_GRAPHS), 1)
    p = (batch_ref[...] == gids).astype(jnp.float32)
    sums = lax.dot_general(p, node, (((0,), (0,)), ((), ())),
                           preferred_element_type=jnp.float32)
    counts = jnp.sum(p, axis=0)
    feats = sums / jnp.maximum(counts, 1.0)[:, None]
    logits = jnp.dot(feats, fcw_ref[...],
                     preferred_element_type=jnp.float32) + fcb_ref[...]
    m = jnp.max(logits, axis=1, keepdims=True)
    lse = jnp.log(jnp.sum(jnp.exp(logits - m), axis=1, keepdims=True)) + m
    out_ref[...] = logits - lse


def _post(a0, a1, den, bias, batch2d, fc_w, fc_b):
    return pl.pallas_call(
        _post_body,
        out_shape=jax.ShapeDtypeStruct((N_GRAPHS, N_CLASSES), jnp.float32),
    )(a0, a1, den, bias.reshape(1, D_HID), batch2d, fc_w,
      fc_b.reshape(1, N_CLASSES))


def kernel(x, edge_index, batch, W, att_src, att_dst, bias, fc_W, fc_b):
    h, av, bv = _pre(x, W, att_src, att_dst)

    loop = jnp.arange(N, dtype=jnp.int32)
    padi = jnp.full((E_PAD - E_RAW,), N, dtype=jnp.int32)
    src_pad = jnp.concatenate([edge_index[0], loop, padi])
    dst_pad = jnp.concatenate([edge_index[1], loop, padi])

    zrow = jnp.zeros((N_ACC - N,), jnp.float32)
    as_pad = jnp.concatenate([av.reshape(-1), zrow])
    ad_pad = jnp.concatenate([bv.reshape(-1), zrow])
    h_pad = jnp.concatenate([h, jnp.zeros((N_ACC - N, D_HID), jnp.float32)])
    zacc = jnp.zeros((N_ACC, D_HID), jnp.float32)

    acc_parts, den_parts = _edge_phase(src_pad, dst_pad, as_pad, ad_pad,
                                       h_pad, zacc)

    return _post(acc_parts[0, :N], acc_parts[1, :N], den_parts[:, :N],
                 bias, batch.reshape(N, 1), fc_W, fc_b)


# speedup vs baseline: 38.0613x; 1.2909x over previous
import jax
import jax.numpy as jnp
from jax import lax
from jax.experimental import pallas as pl
from jax.experimental.pallas import tpu as pltpu
from jax.experimental.pallas import tpu_sc as plsc

N = 10000
E = 320000
D_IN = 128
D_HID = 64
N_GRAPHS = 64
N_CLASSES = 3

NC = 2
NS = 16
NW = NC * NS
LANES = 16

N_ACC = 10112
ROWS_PER_TILE = N_ACC // NS
CHUNK = 128
NB = 4
CHUNKS_PER_TILE = 80
EDGES_PER_TILE = CHUNKS_PER_TILE * CHUNK
E_PAD = NW * EDGES_PER_TILE
N_CHUNKS = E_PAD // CHUNK


def _pre_body(x_ref, w_ref, asrc_ref, adst_ref, h_ref, av_ref, bv_ref):
    h = jnp.dot(x_ref[...], w_ref[...], preferred_element_type=jnp.float32)
    h_ref[pl.ds(0, N), :] = h
    h_ref[pl.ds(N, N_ACC - N), :] = jnp.zeros((N_ACC - N, D_HID), jnp.float32)
    av_ref[pl.ds(0, N), :] = jnp.dot(h, asrc_ref[...],
                                     preferred_element_type=jnp.float32)
    av_ref[pl.ds(N, N_ACC - N), :] = jnp.zeros((N_ACC - N, 1), jnp.float32)
    bv_ref[pl.ds(0, N), :] = jnp.dot(h, adst_ref[...],
                                     preferred_element_type=jnp.float32)
    bv_ref[pl.ds(N, N_ACC - N), :] = jnp.zeros((N_ACC - N, 1), jnp.float32)


def _pre(x, w, att_src, att_dst):
    return pl.pallas_call(
        _pre_body,
        out_shape=(
            jax.ShapeDtypeStruct((N_ACC, D_HID), jnp.float32),
            jax.ShapeDtypeStruct((N_ACC, 1), jnp.float32),
            jax.ShapeDtypeStruct((N_ACC, 1), jnp.float32),
        ),
    )(x, w, att_src.reshape(D_HID, 1), att_dst.reshape(D_HID, 1))


def _edge_body(ei_hbm, as_hbm, ad_hbm, h_hbm, z_hbm,
               acc_out, den_out,
               asv, adv, sd, eav, rows, denv, acc_sh,
               si0, si1, si2, si3, sg0, sg1, sg2, sg3, ss0, ss1, ss2, ss3):
    isems = [si0, si1, si2, si3]
    gsems = [sg0, sg1, sg2, sg3]
    ssems = [ss0, ss1, ss2, ss3]
    c = lax.axis_index("c")
    s = lax.axis_index("s")
    wid = s * NC + c

    rsl = pl.ds(s * ROWS_PER_TILE, ROWS_PER_TILE)
    pltpu.sync_copy(z_hbm, acc_sh.at[rsl])

    pltpu.sync_copy(as_hbm, asv)
    pltpu.sync_copy(ad_hbm, adv)

    def _zden(i, carry):
        denv[pl.ds(i * LANES, LANES)] = jnp.zeros((LANES,), jnp.float32)
        return carry
    lax.fori_loop(0, N_ACC // LANES, _zden, 0)

    plsc.subcore_barrier()

    cid0 = wid * CHUNKS_PER_TILE

    def _idx_start(ci, b):
        pltpu.async_copy(ei_hbm.at[cid0 + ci], sd.at[b], isems[b])

    def _idx_wait(b):
        pltpu.make_async_copy(ei_hbm.at[cid0], sd.at[b], isems[b]).wait()

    def _gather_start(b):
        pltpu.async_copy(h_hbm.at[sd.at[b, 0]], rows.at[b], gsems[b])

    def _gather_wait(b):
        pltpu.make_async_copy(h_hbm.at[sd.at[b, 0]], rows.at[b],
                              gsems[b]).wait()

    def _scatter_start(b):
        pltpu.async_copy(rows.at[b], acc_sh.at[sd.at[b, 1]], ssems[b],
                         add=True)

    def _scatter_wait(b):
        pltpu.make_async_copy(rows.at[b], acc_sh.at[sd.at[b, 1]],
                              ssems[b]).wait()

    for b in range(NB - 1):
        _idx_start(b, b)
    for b in range(NB - 2):
        _idx_wait(b)
        _gather_start(b)

    def _outer(g, carry):
        for b in range(NB):
            ci = g * NB + b

            p3 = (b + NB - 1) % NB

            @pl.when(ci + NB - 1 < CHUNKS_PER_TILE)
            def _():
                @pl.when(ci >= 1)
                def _():
                    _scatter_wait(p3)
                _idx_start(ci + NB - 1, p3)

            p2 = (b + NB - 2) % NB

            @pl.when(ci + NB - 2 < CHUNKS_PER_TILE)
            def _():
                _idx_wait(p2)
                _gather_start(p2)

            def _ea(j, cc):
                sl = pl.ds(j * LANES, LANES)
                sv = sd[b, 0, sl]
                dv = sd[b, 1, sl]
                a = plsc.load_gather(asv, [sv]) + plsc.load_gather(adv, [dv])
                a = jnp.where(a >= 0.0, a, a * jnp.float32(0.2))
                e = jnp.exp(a)
                eav[sl] = e
                plsc.addupdate_scatter(denv, [dv], e)
                return cc
            lax.fori_loop(0, CHUNK // LANES, _ea, 0)

            _gather_wait(b)

            def _scale(r2, cc):
                for u in range(2):
                    w = plsc.load_gather(
                        eav, [jnp.full((LANES,), r2 * 2 + u, jnp.int32)])
                    for k in range(D_HID // LANES):
                        sl = pl.ds(k * LANES, LANES)
                        rows[b, r2 * 2 + u, sl] = rows[b, r2 * 2 + u, sl] * w
                return cc
            lax.fori_loop(0, CHUNK // 2, _scale, 0)

            _scatter_start(b)
        return carry

    lax.fori_loop(0, CHUNKS_PER_TILE // NB, _outer, 0)

    for b in range(NB):
        _scatter_wait(b)

    pltpu.sync_copy(denv, den_out.at[wid])
    plsc.subcore_barrier()
    pltpu.sync_copy(acc_sh.at[rsl], acc_out.at[c, rsl])


def _edge_phase(ei_pack, as_pad, ad_pad, h_pad, zrows):
    k = pl.kernel(
        _edge_body,
        out_type=(
            jax.ShapeDtypeStruct((NC, N_ACC, D_HID), jnp.float32),
            jax.ShapeDtypeStruct((NW, N_ACC), jnp.float32),
        ),
        mesh=plsc.VectorSubcoreMesh(core_axis_name="c", subcore_axis_name="s"),
        compiler_params=pltpu.CompilerParams(needs_layout_passes=False,
                                             use_tc_tiling_on_sc=False),
        scratch_types=[
            pltpu.VMEM((N_ACC,), jnp.float32),
            pltpu.VMEM((N_ACC,), jnp.float32),
            pltpu.VMEM((NB, 2, CHUNK), jnp.int32),
            pltpu.VMEM((CHUNK,), jnp.float32),
            pltpu.VMEM((NB, CHUNK, D_HID), jnp.float32),
            pltpu.VMEM((N_ACC,), jnp.float32),
            pltpu.VMEM_SHARED((N_ACC, D_HID), jnp.float32),
        ] + [pltpu.SemaphoreType.DMA] * 12,
    )
    return k(ei_pack, as_pad, ad_pad, h_pad, zrows)


def _post_body(acc_ref, den_ref, h_ref, av_ref, bv_ref, bias_ref, batch_ref,
               fcw_ref, fcb_ref, out_ref):
    acc = acc_ref[0] + acc_ref[1]
    den = jnp.sum(den_ref[...], axis=0)
    a_self = av_ref[...] + bv_ref[...]
    a_self = jnp.where(a_self >= 0.0, a_self, a_self * 0.2)
    e_self = jnp.exp(a_self)
    h = h_ref[...]
    acc = acc + e_self * h
    den = den + e_self[:, 0]
    node = acc / (den + 1e-16)[:, None] + bias_ref[...]
    gids = lax.broadcasted_iota(jnp.int32, (1, N_GRAPHS), 1)
    p = (batch_ref[...] == gids).astype(jnp.float32)
    sums = lax.dot_general(p, node, (((0,), (0,)), ((), ())),
                           preferred_element_type=jnp.float32)
    counts = jnp.sum(p, axis=0)
    feats = sums / jnp.maximum(counts, 1.0)[:, None]
    logits = jnp.dot(feats, fcw_ref[...],
                     preferred_element_type=jnp.float32) + fcb_ref[...]
    m = jnp.max(logits, axis=1, keepdims=True)
    lse = jnp.log(jnp.sum(jnp.exp(logits - m), axis=1, keepdims=True)) + m
    out_ref[...] = logits - lse


def _post(acc_parts, den_parts, h_pad, av, bv, bias, batch_pad, fc_w, fc_b):
    return pl.pallas_call(
        _post_body,
        out_shape=jax.ShapeDtypeStruct((N_GRAPHS, N_CLASSES), jnp.float32),
    )(acc_parts, den_parts, h_pad, av, bv, bias.reshape(1, D_HID), batch_pad,
      fc_w, fc_b.reshape(1, N_CLASSES))


def kernel(x, edge_index, batch, W, att_src, att_dst, bias, fc_W, fc_b):
    h_pad, av, bv = _pre(x, W, att_src, att_dst)

    padi = jnp.full((E_PAD - E,), N, dtype=jnp.int32)
    src_pad = jnp.concatenate([edge_index[0], padi]).reshape(N_CHUNKS, 1, CHUNK)
    dst_pad = jnp.concatenate([edge_index[1], padi]).reshape(N_CHUNKS, 1, CHUNK)
    ei_pack = jnp.concatenate([src_pad, dst_pad], axis=1)

    zrows = jnp.zeros((ROWS_PER_TILE, D_HID), jnp.float32)

    acc_parts, den_parts = _edge_phase(ei_pack, av.reshape(-1), bv.reshape(-1),
                                       h_pad, zrows)

    batch_pad = jnp.concatenate(
        [batch, jnp.full((N_ACC - N,), -1, jnp.int32)]).reshape(N_ACC, 1)

    return _post(acc_parts, den_parts, h_pad, av, bv, bias, batch_pad,
                 fc_W, fc_b)
